# Initial kernel scaffold; baseline (speedup 1.0000x reference)
import jax
import jax.numpy as jnp
from jax.experimental import pallas as pl


def kernel(weight, input_index):
    raise NotImplementedError("write your pallas kernel here")


# speedup vs baseline: 1.0987x; 1.0987x over previous
import jax
import jax.numpy as jnp
from jax import lax
from jax.experimental import pallas as pl
from jax.experimental.pallas import tpu as pltpu
from jax.experimental.pallas import tpu_sc as plsc

B = 16384 * 50
D = 32
NC, NS = 2, 16
NW = NC * NS
BPW = B // NW
CHUNK = 1280
NCHUNK = BPW // CHUNK

_mesh = plsc.VectorSubcoreMesh(core_axis_name="c", subcore_axis_name="s")


@pl.kernel(
    out_type=jax.ShapeDtypeStruct((B, D), jnp.float32),
    mesh=_mesh,
    scratch_types=[
        pltpu.VMEM((CHUNK,), jnp.int32),
        pltpu.VMEM((CHUNK, D), jnp.float32),
        pltpu.SemaphoreType.DMA,
    ],
    compiler_params=pltpu.CompilerParams(use_tc_tiling_on_sc=False),
)
def _gather(w_hbm, i_hbm, o_hbm, idx_v, rows_v, sem):
    wid = lax.axis_index("s") * NC + lax.axis_index("c")
    base = wid * BPW

    @pl.loop(0, NCHUNK)
    def _(c):
        off = base + c * CHUNK
        pltpu.sync_copy(i_hbm.at[pl.ds(off, CHUNK)], idx_v)
        pltpu.async_copy(w_hbm.at[idx_v], rows_v, sem).wait()
        pltpu.sync_copy(rows_v, o_hbm.at[pl.ds(off, CHUNK)])


@jax.jit
def kernel(weight, input_index):
    flat_idx = input_index.reshape(B)
    out = _gather(weight, flat_idx)
    return out.reshape(*input_index.shape, D)


# speedup vs baseline: 1.1087x; 1.0091x over previous
import jax
import jax.numpy as jnp
from jax import lax
from jax.experimental import pallas as pl
from jax.experimental.pallas import tpu as pltpu
from jax.experimental.pallas import tpu_sc as plsc

B = 16384 * 50
D = 32
NC, NS = 2, 16
NW = NC * NS
BPW = B // NW
CHUNK = 1600
NCHUNK = BPW // CHUNK

_mesh = plsc.VectorSubcoreMesh(core_axis_name="c", subcore_axis_name="s")


@pl.kernel(
    out_type=jax.ShapeDtypeStruct((B, D), jnp.float32),
    mesh=_mesh,
    scratch_types=[
        pltpu.VMEM((CHUNK,), jnp.int32),
        pltpu.VMEM((CHUNK,), jnp.int32),
        pltpu.VMEM((CHUNK, D), jnp.float32),
        pltpu.VMEM((CHUNK, D), jnp.float32),
        pltpu.SemaphoreType.DMA,
        pltpu.SemaphoreType.DMA,
    ],
    compiler_params=pltpu.CompilerParams(use_tc_tiling_on_sc=False),
)
def _gather(w_hbm, i_hbm, o_hbm, idx0, idx1, rows0, rows1, sem0, sem1):
    wid = lax.axis_index("s") * NC + lax.axis_index("c")
    base = wid * BPW

    def chunk_off(c):
        return base + c * CHUNK

    pltpu.sync_copy(i_hbm.at[pl.ds(chunk_off(0), CHUNK)], idx0)
    pltpu.async_copy(w_hbm.at[idx0], rows0, sem0)

    @pl.loop(0, NCHUNK, step=2)
    def _(c):
        pltpu.sync_copy(i_hbm.at[pl.ds(chunk_off(c + 1), CHUNK)], idx1)
        pltpu.async_copy(w_hbm.at[idx1], rows1, sem1)
        pltpu.make_async_copy(w_hbm.at[idx0], rows0, sem0).wait()
        pltpu.sync_copy(rows0, o_hbm.at[pl.ds(chunk_off(c), CHUNK)])

        @pl.when(c + 2 < NCHUNK)
        def _():
            pltpu.sync_copy(i_hbm.at[pl.ds(chunk_off(c + 2), CHUNK)], idx0)
            pltpu.async_copy(w_hbm.at[idx0], rows0, sem0)

        pltpu.make_async_copy(w_hbm.at[idx1], rows1, sem1).wait()
        pltpu.sync_copy(rows1, o_hbm.at[pl.ds(chunk_off(c + 1), CHUNK)])


@jax.jit
def kernel(weight, input_index):
    flat_idx = input_index.reshape(B)
    out = _gather(weight, flat_idx)
    return out.reshape(*input_index.shape, D)


# speedup vs baseline: 1.1126x; 1.0035x over previous
import jax
import jax.numpy as jnp
from jax import lax
from jax.experimental import pallas as pl
from jax.experimental.pallas import tpu as pltpu
from jax.experimental.pallas import tpu_sc as plsc

B = 16384 * 50
D = 32
NC, NS = 2, 16
NW = NC * NS
BPW = B // NW
CHUNK = 800
NCHUNK = BPW // CHUNK
NBUF = 4

_mesh = plsc.VectorSubcoreMesh(core_axis_name="c", subcore_axis_name="s")


@pl.kernel(
    out_type=jax.ShapeDtypeStruct((B, D), jnp.float32),
    mesh=_mesh,
    scratch_types=[
        pltpu.VMEM((BPW,), jnp.int32),
        [pltpu.VMEM((CHUNK, D), jnp.float32) for _ in range(NBUF)],
        [pltpu.SemaphoreType.DMA for _ in range(NBUF)],
    ],
    compiler_params=pltpu.CompilerParams(use_tc_tiling_on_sc=False),
)
def _gather(w_hbm, i_hbm, o_hbm, idx_all, rows, sems):
    wid = lax.axis_index("s") * NC + lax.axis_index("c")
    base = wid * BPW

    pltpu.sync_copy(i_hbm.at[pl.ds(base, BPW)], idx_all)

    def idx_of(c):
        return idx_all.at[pl.ds(c * CHUNK, CHUNK)]

    for b in range(NBUF):
        pltpu.async_copy(w_hbm.at[idx_of(b)], rows[b], sems[b])

    @pl.loop(0, NCHUNK, step=NBUF)
    def _(c):
        for b in range(NBUF):
            pltpu.make_async_copy(w_hbm.at[idx_of(c + b)], rows[b], sems[b]).wait()
            pltpu.sync_copy(rows[b], o_hbm.at[pl.ds(base + (c + b) * CHUNK, CHUNK)])

            @pl.when(c + b + NBUF < NCHUNK)
            def _():
                pltpu.async_copy(w_hbm.at[idx_of(c + b + NBUF)], rows[b], sems[b])


@jax.jit
def kernel(weight, input_index):
    flat_idx = input_index.reshape(B)
    out = _gather(weight, flat_idx)
    return out.reshape(*input_index.shape, D)
